# baseline (speedup 1.0000x reference)
import functools

import jax
import jax.numpy as jnp
from jax import lax
from jax.experimental import pallas as pl
from jax.experimental.pallas import tpu as pltpu
from jax.experimental.pallas import tpu_sc as plsc

N = 10000
E = 160000
D = 256
H = 256
O = 64
NP = 10240
NC = 2
NS = 16
HH = H // 2
NH = NP // 2
GR = NH
AR = NH + 8
SENT = NP - 1

KB = 128
EP = 163840
NB1 = EP // (NC * NS) // KB
NB2 = EP // NS // KB
NSLOT = 3
RPT = NP // NS
ZR = NH // NS

_mesh = plsc.VectorSubcoreMesh(
    core_axis_name="c", subcore_axis_name="s", num_cores=NC, num_subcores=NS
)


def _zero_1d(ref, n):
    def body(i, _):
        ref[pl.ds(i * 16, 16)] = jnp.zeros((16,), jnp.float32)
        return 0
    lax.fori_loop(0, n // 16, body, 0)


def _zero_2d(ref, rows, cols):
    def body(i, _):
        r = i // (cols // 16)
        j = i % (cols // 16)
        ref[r, pl.ds(j * 16, 16)] = jnp.zeros((16,), jnp.float32)
        return 0
    lax.fori_loop(0, rows * (cols // 16), body, 0)


@functools.partial(
    pl.kernel,
    out_type=[
        jax.ShapeDtypeStruct((NP,), jnp.float32),
        jax.ShapeDtypeStruct((NP,), jnp.float32),
    ],
    mesh=_mesh,
    scratch_types=[
        pltpu.VMEM((NB1, KB), jnp.int32),
        pltpu.VMEM((KB,), jnp.float32),
        pltpu.VMEM((RPT,), jnp.float32),
        pltpu.VMEM_SHARED((NP,), jnp.float32),
    ],
)
def _deg_kernel(dst2d, cnt0, cnt1, dstb, ones, zbuf, acc):
    c = lax.axis_index("c")
    s = lax.axis_index("s")
    _zero_1d(zbuf, RPT)
    for j in range(8):
        ones[pl.ds(j * 16, 16)] = jnp.ones((16,), jnp.float32)
    pltpu.sync_copy(zbuf, acc.at[pl.ds(s * RPT, RPT)])
    row0 = (c * NS + s) * NB1
    pltpu.sync_copy(dst2d.at[pl.ds(row0, NB1)], dstb)
    plsc.subcore_barrier()

    def body(i, _):
        pltpu.sync_copy(ones, acc.at[dstb.at[i]], add=True)
        return 0

    lax.fori_loop(0, NB1, body, 0)
    plsc.subcore_barrier()

    @pl.when(c == 0)
    def _():
        pltpu.sync_copy(acc.at[pl.ds(s * RPT, RPT)], cnt0.at[pl.ds(s * RPT, RPT)])

    @pl.when(c == 1)
    def _():
        pltpu.sync_copy(acc.at[pl.ds(s * RPT, RPT)], cnt1.at[pl.ds(s * RPT, RPT)])


@functools.partial(
    pl.kernel,
    out_type=[
        jax.ShapeDtypeStruct((NP,), jnp.float32),
        jax.ShapeDtypeStruct((NP,), jnp.float32),
    ],
    mesh=_mesh,
    scratch_types=[
        pltpu.VMEM((NB1, KB), jnp.int32),
        pltpu.VMEM((NB1, KB), jnp.int32),
        pltpu.VMEM((2, KB), jnp.float32),
        pltpu.VMEM((RPT,), jnp.float32),
        pltpu.VMEM_SHARED((NP,), jnp.float32),
        pltpu.SemaphoreType.DMA((2,)),
    ],
)
def _cprime_kernel(src2d, dst2d, dinvp, outC0, outC1,
                   srcb, dstb, dbuf, zbuf, accc, gsem):
    c = lax.axis_index("c")
    s = lax.axis_index("s")
    _zero_1d(zbuf, RPT)
    pltpu.sync_copy(zbuf, accc.at[pl.ds(s * RPT, RPT)])
    row0 = (c * NS + s) * NB1
    pltpu.sync_copy(src2d.at[pl.ds(row0, NB1)], srcb)
    pltpu.sync_copy(dst2d.at[pl.ds(row0, NB1)], dstb)
    plsc.subcore_barrier()

    def gissue(i, slot):
        pltpu.async_copy(dinvp.at[dstb.at[i]], dbuf.at[slot], gsem.at[slot])

    def gwait(i, slot):
        pltpu.make_async_copy(
            dinvp.at[dstb.at[i]], dbuf.at[slot], gsem.at[slot]).wait()

    gissue(0, 0)

    def body(i, _):
        slot = lax.rem(i, 2)
        other = 1 - slot
        gwait(i, slot)

        @pl.when(i + 1 < NB1)
        def _():
            gissue(i + 1, other)

        pltpu.sync_copy(dbuf.at[slot], accc.at[srcb.at[i]], add=True)
        return 0

    lax.fori_loop(0, NB1, body, 0)
    plsc.subcore_barrier()

    @pl.when(c == 0)
    def _():
        pltpu.sync_copy(accc.at[pl.ds(s * RPT, RPT)], outC0.at[pl.ds(s * RPT, RPT)])

    @pl.when(c == 1)
    def _():
        pltpu.sync_copy(accc.at[pl.ds(s * RPT, RPT)], outC1.at[pl.ds(s * RPT, RPT)])


EPT = EP // NS


@functools.partial(
    pl.kernel,
    out_type=[
        jax.ShapeDtypeStruct((NP, HH), jnp.float32),
        jax.ShapeDtypeStruct((NP, HH), jnp.float32),
    ],
    mesh=_mesh,
    scratch_types=[
        pltpu.VMEM((NB2, KB), jnp.int32),
        pltpu.VMEM((NB2, KB), jnp.int32),
        pltpu.VMEM((NB2, KB), jnp.int32),
        pltpu.VMEM((NB2, KB), jnp.int32),
        pltpu.VMEM((2, KB, HH), jnp.float32),
        pltpu.VMEM((32, HH), jnp.float32),
        pltpu.VMEM((RPT,), jnp.float32),
        pltpu.VMEM((32,), jnp.int32),
        pltpu.VMEM_SHARED((AR, HH), jnp.float32),
        pltpu.SemaphoreType.DMA((2,)),
    ],
    compiler_params=pltpu.CompilerParams(needs_layout_passes=False),
)
def _edge_kernel(src2d, dst2d, glo, ghi,
                 out_lo, out_hi,
                 srcA2, dstA2, srcB2, dstB2,
                 rows, zbuf2, zbuf1, cntbuf, acc, gsem):
    del cntbuf
    c = lax.axis_index("c")
    s = lax.axis_index("s")
    _zero_2d(zbuf2, 32, HH)
    _zero_1d(zbuf1, RPT)

    def zero_acc():
        for k in range(ZR // 32):
            pltpu.sync_copy(zbuf2, acc.at[pl.ds(s * ZR + k * 32, 32)])

        @pl.when(s == 0)
        def _():
            pltpu.sync_copy(zbuf2.at[pl.ds(0, 8)], acc.at[pl.ds(NH, 8)])

    def scatter_loop(gtab, srcl, dstl, nb):
        def gissue(i, slot):
            pltpu.async_copy(gtab.at[srcl.at[i]], rows.at[slot], gsem.at[slot])

        def gwait(i, slot):
            pltpu.make_async_copy(
                gtab.at[srcl.at[i]], rows.at[slot], gsem.at[slot]).wait()

        @pl.when(nb >= 1)
        def _():
            gissue(0, 0)

        def body(i, _):
            @pl.when(i < nb)
            def _():
                slot = lax.rem(i, 2)
                other = 1 - slot
                gwait(i, slot)

                @pl.when(i + 1 < nb)
                def _():
                    gissue(i + 1, other)

                pltpu.sync_copy(rows.at[slot], acc.at[dstl.at[i]], add=True)
            return 0

        lax.fori_loop(0, NB2, body, 0)

    def copy_out(dest, half):
        pltpu.sync_copy(acc.at[pl.ds(s * ZR, ZR)],
                        dest.at[pl.ds(half * NH + s * ZR, ZR)])

    zero_acc()
    pltpu.sync_copy(src2d.at[pl.ds(s * NB2, NB2)], srcA2)
    pltpu.sync_copy(dst2d.at[pl.ds(s * NB2, NB2)], dstA2)

    def part(v, carry):
        cntAv, cntBv = carry
        r0 = v // (KB // 16)
        c0 = (v % (KB // 16)) * 16
        sv = srcA2[r0, pl.ds(c0, 16)]
        dv = dstA2[r0, pl.ds(c0, 16)]
        mA = dv < NH
        iA = mA.astype(jnp.int32)
        iB = 1 - iA
        csA = plsc.cumsum(iA)
        csB = plsc.cumsum(iB)
        posA = cntAv + csA - 1
        posB = cntBv + csB - 1
        rowA = lax.shift_right_logical(posA, 7)
        colA = lax.bitwise_and(posA, 127)
        rowB = lax.shift_right_logical(posB, 7)
        colB = lax.bitwise_and(posB, 127)
        mB = jnp.logical_not(mA)
        plsc.store_scatter(srcA2, [rowA, colA], sv, mask=mA)
        plsc.store_scatter(dstA2, [rowA, colA], dv, mask=mA)
        plsc.store_scatter(srcB2, [rowB, colB], sv, mask=mB)
        plsc.store_scatter(dstB2, [rowB, colB], dv - NH, mask=mB)
        totA = plsc.cummax(lax.rev(csA, (0,)))
        totB = plsc.cummax(lax.rev(csB, (0,)))
        return (cntAv + totA, cntBv + totB)

    zvec = jnp.zeros((16,), jnp.int32)
    cntAv, cntBv = lax.fori_loop(0, EPT // 16, part, (zvec, zvec))

    iota16 = lax.iota(jnp.int32, 16)
    grv = jnp.full((16,), GR, jnp.int32)
    zv16 = jnp.zeros((16,), jnp.int32)

    def tail_fill(dstl, srcl, cntv):
        base = lax.shift_left(lax.shift_right_logical(cntv, 4), 4)
        endv = lax.shift_left(
            lax.shift_right_logical(cntv + 127, 7), 7)
        for k in range(9):
            pos = base + k * 16 + iota16
            m = jnp.logical_and(pos >= cntv, pos < endv)
            idx = [lax.shift_right_logical(pos, 7), lax.bitwise_and(pos, 127)]
            plsc.store_scatter(dstl, idx, grv, mask=m)
            if srcl is not None:
                plsc.store_scatter(srcl, idx, zv16, mask=m)

    tail_fill(dstA2, None, cntAv)
    tail_fill(dstB2, srcB2, cntBv)

    nbA = lax.shift_right_logical(cntAv[0] + 127, 7)
    nbB = lax.shift_right_logical(cntBv[0] + 127, 7)
    plsc.subcore_barrier()

    @pl.when(c == 0)
    def _():
        scatter_loop(glo, srcA2, dstA2, nbA)

    @pl.when(c == 1)
    def _():
        scatter_loop(ghi, srcA2, dstA2, nbA)

    plsc.subcore_barrier()

    @pl.when(c == 0)
    def _():
        copy_out(out_lo, 0)

    @pl.when(c == 1)
    def _():
        copy_out(out_hi, 0)

    plsc.subcore_barrier()
    zero_acc()
    plsc.subcore_barrier()

    @pl.when(c == 0)
    def _():
        scatter_loop(glo, srcB2, dstB2, nbB)

    @pl.when(c == 1)
    def _():
        scatter_loop(ghi, srcB2, dstB2, nbB)

    plsc.subcore_barrier()

    @pl.when(c == 0)
    def _():
        copy_out(out_lo, 1)

    @pl.when(c == 1)
    def _():
        copy_out(out_hi, 1)


BN = 1000
NBLK = N // BN
EB = 128


def _mm_body(x_ref, w_ref, c0_ref, c1_ref,
             glo_ref, ghi_ref, dv_ref, dq_ref):
    deg = c0_ref[...] + c1_ref[...] + 1.0
    dv = lax.rsqrt(deg)
    h = jnp.dot(x_ref[...], w_ref[...], preferred_element_type=jnp.float32,
                precision=lax.Precision.HIGHEST)
    g = h * dv
    glo_ref[...] = g[:, :HH]
    ghi_ref[...] = g[:, HH:]
    dv_ref[...] = dv
    dq_ref[...] = 1.0 / deg


def _mm_call(x, W1, cnt0, cnt1):
    vspec = pl.BlockSpec((BN, 1), lambda i: (i, 0))
    return pl.pallas_call(
        _mm_body,
        grid=(NBLK,),
        in_specs=[
            pl.BlockSpec((BN, D), lambda i: (i, 0)),
            pl.BlockSpec((D, H), lambda i: (0, 0)),
            vspec,
            vspec,
        ],
        out_specs=[
            pl.BlockSpec((BN, HH), lambda i: (i, 0)),
            pl.BlockSpec((BN, HH), lambda i: (i, 0)),
            vspec,
            vspec,
        ],
        out_shape=[
            jax.ShapeDtypeStruct((NP, HH), jnp.float32),
            jax.ShapeDtypeStruct((NP, HH), jnp.float32),
            jax.ShapeDtypeStruct((N, 1), jnp.float32),
            jax.ShapeDtypeStruct((N, 1), jnp.float32),
        ],
    )(x, W1, cnt0, cnt1)


def _fin_body(a0_ref, a1_ref, g0_ref, g1_ref, dv_ref, dq_ref, cp0_ref, cp1_ref,
              b1_ref, w2_ref, b2_ref, wl_ref, bl_ref,
              out_ref, vacc):
    i = pl.program_id(0)

    @pl.when(i == 0)
    def _():
        vacc[...] = jnp.zeros((2, HH), jnp.float32)

    dv = dv_ref[...]
    cv = dv * (cp0_ref[...] + cp1_ref[...]) + dq_ref[...]
    aq = (a0_ref, a1_ref)
    gq = (g0_ref, g1_ref)
    for q in range(2):
        r = jnp.maximum(dv * (aq[q][...] + gq[q][...]) + b1_ref[q:q + 1, :], 0.0)
        vacc[q:q + 1, :] += jnp.sum(r * cv, axis=0, keepdims=True)

    @pl.when(i == NBLK - 1)
    def _():
        mean = b2_ref[...] * 0.0
        for q in range(2):
            mean += jnp.dot(vacc[q:q + 1, :], w2_ref[q],
                            preferred_element_type=jnp.float32,
                            precision=lax.Precision.HIGHEST)
        mean = mean * (1.0 / N) + b2_ref[...]
        out_ref[...] = jnp.dot(mean, wl_ref[...], preferred_element_type=jnp.float32,
                               precision=lax.Precision.HIGHEST) + bl_ref[...]


def _fin_call(a0, a1, g0, g1, dinv, dinvsq, cp0, cp1, b1r, W2r, b2, Wl, bl):
    hspec = pl.BlockSpec((BN, HH), lambda i: (i, 0))
    vspec = pl.BlockSpec((BN, 1), lambda i: (i, 0))
    return pl.pallas_call(
        _fin_body,
        grid=(NBLK,),
        in_specs=[
            hspec, hspec, hspec, hspec,
            vspec, vspec, vspec, vspec,
            pl.BlockSpec((2, HH), lambda i: (0, 0)),
            pl.BlockSpec((2, HH, H), lambda i: (0, 0, 0)),
            pl.BlockSpec((1, H), lambda i: (0, 0)),
            pl.BlockSpec((H, O), lambda i: (0, 0)),
            pl.BlockSpec((1, O), lambda i: (0, 0)),
        ],
        out_specs=pl.BlockSpec((1, O), lambda i: (0, 0)),
        out_shape=jax.ShapeDtypeStruct((1, O), jnp.float32),
        scratch_shapes=[pltpu.VMEM((2, HH), jnp.float32)],
    )(a0, a1, g0, g1, dinv, dinvsq, cp0, cp1, b1r, W2r, b2, Wl, bl)


@jax.jit
def kernel(x, edge_index, W1, b1, W2, b2, W_lin, b_lin):
    ei = edge_index.astype(jnp.int32)
    src = ei[0]
    dst = ei[1]
    pad = EP - E
    src_p = jnp.concatenate([src, jnp.zeros((pad,), jnp.int32)])
    dst_p = jnp.concatenate([dst, jnp.full((pad,), SENT, jnp.int32)])
    dst2d_p = dst_p.reshape(EP // KB, KB)
    cnt0, cnt1 = _deg_kernel(dst2d_p)
    glo, ghi, dinv, dinvsq = _mm_call(
        x, W1, cnt0.reshape(NP, 1), cnt1.reshape(NP, 1),
    )
    dinvp = jnp.pad(dinv.reshape(N), (0, NP - N))
    cp0, cp1 = _cprime_kernel(src_p.reshape(EP // KB, KB), dst2d_p, dinvp)
    a_lo, a_hi = _edge_kernel(src_p.reshape(EP // KB, KB), dst2d_p, glo, ghi)
    return _fin_call(
        a_lo, a_hi, glo, ghi, dinv, dinvsq,
        cp0.reshape(NP, 1), cp1.reshape(NP, 1),
        b1.reshape(2, HH), W2.reshape(2, HH, H),
        b2.reshape(1, H), W_lin, b_lin.reshape(1, O),
    )


# speedup vs baseline: 1.0599x; 1.0599x over previous
import functools

import jax
import jax.numpy as jnp
from jax import lax
from jax.experimental import pallas as pl
from jax.experimental.pallas import tpu as pltpu
from jax.experimental.pallas import tpu_sc as plsc

N = 10000
E = 160000
D = 256
H = 256
O = 64
NP = 10240
NC = 2
NS = 16
HH = H // 2
NH = NP // 2
GR = NH
AR = NH + 8
SENT = NP - 1

KB = 128
EP = 163840
NB1 = EP // (NC * NS) // KB
NB2 = EP // NS // KB
NSLOT = 3
RPT = NP // NS
ZR = NH // NS

_mesh = plsc.VectorSubcoreMesh(
    core_axis_name="c", subcore_axis_name="s", num_cores=NC, num_subcores=NS
)


def _zero_1d(ref, n):
    def body(i, _):
        ref[pl.ds(i * 16, 16)] = jnp.zeros((16,), jnp.float32)
        return 0
    lax.fori_loop(0, n // 16, body, 0)


def _zero_2d(ref, rows, cols):
    def body(i, _):
        r = i // (cols // 16)
        j = i % (cols // 16)
        ref[r, pl.ds(j * 16, 16)] = jnp.zeros((16,), jnp.float32)
        return 0
    lax.fori_loop(0, rows * (cols // 16), body, 0)


@functools.partial(
    pl.kernel,
    out_type=[
        jax.ShapeDtypeStruct((NP,), jnp.float32),
        jax.ShapeDtypeStruct((NP,), jnp.float32),
    ],
    mesh=_mesh,
    scratch_types=[
        pltpu.VMEM((NB1, KB), jnp.int32),
        pltpu.VMEM((KB,), jnp.float32),
        pltpu.VMEM((RPT,), jnp.float32),
        pltpu.VMEM_SHARED((NP,), jnp.float32),
    ],
)
def _deg_kernel(dst2d, cnt0, cnt1, dstb, ones, zbuf, acc):
    c = lax.axis_index("c")
    s = lax.axis_index("s")
    _zero_1d(zbuf, RPT)
    for j in range(8):
        ones[pl.ds(j * 16, 16)] = jnp.ones((16,), jnp.float32)
    pltpu.sync_copy(zbuf, acc.at[pl.ds(s * RPT, RPT)])
    row0 = (c * NS + s) * NB1
    pltpu.sync_copy(dst2d.at[pl.ds(row0, NB1)], dstb)
    plsc.subcore_barrier()

    def body(i, _):
        pltpu.sync_copy(ones, acc.at[dstb.at[i]], add=True)
        return 0

    lax.fori_loop(0, NB1, body, 0)
    plsc.subcore_barrier()

    @pl.when(c == 0)
    def _():
        pltpu.sync_copy(acc.at[pl.ds(s * RPT, RPT)], cnt0.at[pl.ds(s * RPT, RPT)])

    @pl.when(c == 1)
    def _():
        pltpu.sync_copy(acc.at[pl.ds(s * RPT, RPT)], cnt1.at[pl.ds(s * RPT, RPT)])


@functools.partial(
    pl.kernel,
    out_type=[
        jax.ShapeDtypeStruct((NP,), jnp.float32),
        jax.ShapeDtypeStruct((NP,), jnp.float32),
    ],
    mesh=_mesh,
    scratch_types=[
        pltpu.VMEM((NB1, KB), jnp.int32),
        pltpu.VMEM((NB1, KB), jnp.int32),
        pltpu.VMEM((2, KB), jnp.float32),
        pltpu.VMEM((RPT,), jnp.float32),
        pltpu.VMEM_SHARED((NP,), jnp.float32),
        pltpu.SemaphoreType.DMA((2,)),
    ],
)
def _cprime_kernel(src2d, dst2d, dinvp, outC0, outC1,
                   srcb, dstb, dbuf, zbuf, accc, gsem):
    c = lax.axis_index("c")
    s = lax.axis_index("s")
    _zero_1d(zbuf, RPT)
    pltpu.sync_copy(zbuf, accc.at[pl.ds(s * RPT, RPT)])
    row0 = (c * NS + s) * NB1
    pltpu.sync_copy(src2d.at[pl.ds(row0, NB1)], srcb)
    pltpu.sync_copy(dst2d.at[pl.ds(row0, NB1)], dstb)
    plsc.subcore_barrier()

    def gissue(i, slot):
        pltpu.async_copy(dinvp.at[dstb.at[i]], dbuf.at[slot], gsem.at[slot])

    def gwait(i, slot):
        pltpu.make_async_copy(
            dinvp.at[dstb.at[i]], dbuf.at[slot], gsem.at[slot]).wait()

    gissue(0, 0)

    def body(i, _):
        slot = lax.rem(i, 2)
        other = 1 - slot
        gwait(i, slot)

        @pl.when(i + 1 < NB1)
        def _():
            gissue(i + 1, other)

        pltpu.sync_copy(dbuf.at[slot], accc.at[srcb.at[i]], add=True)
        return 0

    lax.fori_loop(0, NB1, body, 0)
    plsc.subcore_barrier()

    @pl.when(c == 0)
    def _():
        pltpu.sync_copy(accc.at[pl.ds(s * RPT, RPT)], outC0.at[pl.ds(s * RPT, RPT)])

    @pl.when(c == 1)
    def _():
        pltpu.sync_copy(accc.at[pl.ds(s * RPT, RPT)], outC1.at[pl.ds(s * RPT, RPT)])


EPT = EP // NS


@functools.partial(
    pl.kernel,
    out_type=[
        jax.ShapeDtypeStruct((NP, HH), jnp.float32),
        jax.ShapeDtypeStruct((NP, HH), jnp.float32),
    ],
    mesh=_mesh,
    scratch_types=[
        pltpu.VMEM((NB2, KB), jnp.int32),
        pltpu.VMEM((NB2, KB), jnp.int32),
        pltpu.VMEM((NB2, KB), jnp.int32),
        pltpu.VMEM((NB2, KB), jnp.int32),
        pltpu.VMEM((2, KB, HH), jnp.float32),
        pltpu.VMEM((32, HH), jnp.float32),
        pltpu.VMEM((RPT,), jnp.float32),
        pltpu.VMEM((32,), jnp.int32),
        pltpu.VMEM_SHARED((AR, HH), jnp.float32),
        pltpu.SemaphoreType.DMA((2,)),
        pltpu.SemaphoreType.DMA((2,)),
    ],
    compiler_params=pltpu.CompilerParams(needs_layout_passes=False),
)
def _edge_kernel(src2d, dst2d, glo, ghi,
                 out_lo, out_hi,
                 srcA2, dstA2, srcB2, dstB2,
                 rows, zbuf2, zbuf1, cntbuf, acc, gsem, ssem):
    del cntbuf
    c = lax.axis_index("c")
    s = lax.axis_index("s")
    _zero_2d(zbuf2, 32, HH)
    _zero_1d(zbuf1, RPT)

    def zero_acc():
        for k in range(ZR // 32):
            pltpu.sync_copy(zbuf2, acc.at[pl.ds(s * ZR + k * 32, 32)])

        @pl.when(s == 0)
        def _():
            pltpu.sync_copy(zbuf2.at[pl.ds(0, 8)], acc.at[pl.ds(NH, 8)])

    def scatter_loop(gtab, srcl, dstl, nb):
        def gissue(i, slot):
            pltpu.async_copy(gtab.at[srcl.at[i]], rows.at[slot], gsem.at[slot])

        def gwait(i, slot):
            pltpu.make_async_copy(
                gtab.at[srcl.at[i]], rows.at[slot], gsem.at[slot]).wait()

        def sissue(i, slot):
            pltpu.async_copy(rows.at[slot], acc.at[dstl.at[i]],
                             ssem.at[slot], add=True)

        def swait(i, slot):
            pltpu.make_async_copy(
                rows.at[slot], acc.at[dstl.at[i]], ssem.at[slot]).wait()

        @pl.when(nb >= 1)
        def _():
            gissue(0, 0)

        def body(i, _):
            @pl.when(i < nb)
            def _():
                slot = lax.rem(i, 2)
                other = 1 - slot

                @pl.when(i >= 1)
                def _():
                    swait(i - 1, other)

                @pl.when(i + 1 < nb)
                def _():
                    gissue(i + 1, other)

                gwait(i, slot)
                sissue(i, slot)
            return 0

        lax.fori_loop(0, NB2, body, 0)

        @pl.when(nb >= 1)
        def _():
            swait(nb - 1, lax.rem(nb - 1, 2))

    def copy_out(dest, half):
        pltpu.sync_copy(acc.at[pl.ds(s * ZR, ZR)],
                        dest.at[pl.ds(half * NH + s * ZR, ZR)])

    zero_acc()
    pltpu.sync_copy(src2d.at[pl.ds(s * NB2, NB2)], srcA2)
    pltpu.sync_copy(dst2d.at[pl.ds(s * NB2, NB2)], dstA2)

    def part(v, carry):
        cntAv, cntBv = carry
        r0 = v // (KB // 16)
        c0 = (v % (KB // 16)) * 16
        sv = srcA2[r0, pl.ds(c0, 16)]
        dv = dstA2[r0, pl.ds(c0, 16)]
        mA = dv < NH
        iA = mA.astype(jnp.int32)
        iB = 1 - iA
        csA = plsc.cumsum(iA)
        csB = plsc.cumsum(iB)
        posA = cntAv + csA - 1
        posB = cntBv + csB - 1
        rowA = lax.shift_right_logical(posA, 7)
        colA = lax.bitwise_and(posA, 127)
        rowB = lax.shift_right_logical(posB, 7)
        colB = lax.bitwise_and(posB, 127)
        mB = jnp.logical_not(mA)
        plsc.store_scatter(srcA2, [rowA, colA], sv, mask=mA)
        plsc.store_scatter(dstA2, [rowA, colA], dv, mask=mA)
        plsc.store_scatter(srcB2, [rowB, colB], sv, mask=mB)
        plsc.store_scatter(dstB2, [rowB, colB], dv - NH, mask=mB)
        totA = plsc.cummax(lax.rev(csA, (0,)))
        totB = plsc.cummax(lax.rev(csB, (0,)))
        return (cntAv + totA, cntBv + totB)

    zvec = jnp.zeros((16,), jnp.int32)
    cntAv, cntBv = lax.fori_loop(0, EPT // 16, part, (zvec, zvec))

    iota16 = lax.iota(jnp.int32, 16)
    grv = jnp.full((16,), GR, jnp.int32)
    zv16 = jnp.zeros((16,), jnp.int32)

    def tail_fill(dstl, srcl, cntv):
        base = lax.shift_left(lax.shift_right_logical(cntv, 4), 4)
        endv = lax.shift_left(
            lax.shift_right_logical(cntv + 127, 7), 7)
        for k in range(9):
            pos = base + k * 16 + iota16
            m = jnp.logical_and(pos >= cntv, pos < endv)
            idx = [lax.shift_right_logical(pos, 7), lax.bitwise_and(pos, 127)]
            plsc.store_scatter(dstl, idx, grv, mask=m)
            if srcl is not None:
                plsc.store_scatter(srcl, idx, zv16, mask=m)

    tail_fill(dstA2, None, cntAv)
    tail_fill(dstB2, srcB2, cntBv)

    nbA = lax.shift_right_logical(cntAv[0] + 127, 7)
    nbB = lax.shift_right_logical(cntBv[0] + 127, 7)
    plsc.subcore_barrier()

    @pl.when(c == 0)
    def _():
        scatter_loop(glo, srcA2, dstA2, nbA)

    @pl.when(c == 1)
    def _():
        scatter_loop(ghi, srcA2, dstA2, nbA)

    plsc.subcore_barrier()

    @pl.when(c == 0)
    def _():
        copy_out(out_lo, 0)

    @pl.when(c == 1)
    def _():
        copy_out(out_hi, 0)

    plsc.subcore_barrier()
    zero_acc()
    plsc.subcore_barrier()

    @pl.when(c == 0)
    def _():
        scatter_loop(glo, srcB2, dstB2, nbB)

    @pl.when(c == 1)
    def _():
        scatter_loop(ghi, srcB2, dstB2, nbB)

    plsc.subcore_barrier()

    @pl.when(c == 0)
    def _():
        copy_out(out_lo, 1)

    @pl.when(c == 1)
    def _():
        copy_out(out_hi, 1)


BN = 1000
NBLK = N // BN
EB = 128


def _mm_body(x_ref, w_ref, c0_ref, c1_ref,
             glo_ref, ghi_ref, dv_ref, dq_ref):
    deg = c0_ref[...] + c1_ref[...] + 1.0
    dv = lax.rsqrt(deg)
    h = jnp.dot(x_ref[...], w_ref[...], preferred_element_type=jnp.float32,
                precision=lax.Precision.HIGHEST)
    g = h * dv
    glo_ref[...] = g[:, :HH]
    ghi_ref[...] = g[:, HH:]
    dv_ref[...] = dv
    dq_ref[...] = 1.0 / deg


def _mm_call(x, W1, cnt0, cnt1):
    vspec = pl.BlockSpec((BN, 1), lambda i: (i, 0))
    return pl.pallas_call(
        _mm_body,
        grid=(NBLK,),
        in_specs=[
            pl.BlockSpec((BN, D), lambda i: (i, 0)),
            pl.BlockSpec((D, H), lambda i: (0, 0)),
            vspec,
            vspec,
        ],
        out_specs=[
            pl.BlockSpec((BN, HH), lambda i: (i, 0)),
            pl.BlockSpec((BN, HH), lambda i: (i, 0)),
            vspec,
            vspec,
        ],
        out_shape=[
            jax.ShapeDtypeStruct((NP, HH), jnp.float32),
            jax.ShapeDtypeStruct((NP, HH), jnp.float32),
            jax.ShapeDtypeStruct((N, 1), jnp.float32),
            jax.ShapeDtypeStruct((N, 1), jnp.float32),
        ],
    )(x, W1, cnt0, cnt1)


def _fin_body(a0_ref, a1_ref, g0_ref, g1_ref, dv_ref, dq_ref, cp0_ref, cp1_ref,
              b1_ref, w2_ref, b2_ref, wl_ref, bl_ref,
              out_ref, vacc):
    i = pl.program_id(0)

    @pl.when(i == 0)
    def _():
        vacc[...] = jnp.zeros((2, HH), jnp.float32)

    dv = dv_ref[...]
    cv = dv * (cp0_ref[...] + cp1_ref[...]) + dq_ref[...]
    aq = (a0_ref, a1_ref)
    gq = (g0_ref, g1_ref)
    for q in range(2):
        r = jnp.maximum(dv * (aq[q][...] + gq[q][...]) + b1_ref[q:q + 1, :], 0.0)
        vacc[q:q + 1, :] += jnp.sum(r * cv, axis=0, keepdims=True)

    @pl.when(i == NBLK - 1)
    def _():
        mean = b2_ref[...] * 0.0
        for q in range(2):
            mean += jnp.dot(vacc[q:q + 1, :], w2_ref[q],
                            preferred_element_type=jnp.float32,
                            precision=lax.Precision.HIGHEST)
        mean = mean * (1.0 / N) + b2_ref[...]
        out_ref[...] = jnp.dot(mean, wl_ref[...], preferred_element_type=jnp.float32,
                               precision=lax.Precision.HIGHEST) + bl_ref[...]


def _fin_call(a0, a1, g0, g1, dinv, dinvsq, cp0, cp1, b1r, W2r, b2, Wl, bl):
    hspec = pl.BlockSpec((BN, HH), lambda i: (i, 0))
    vspec = pl.BlockSpec((BN, 1), lambda i: (i, 0))
    return pl.pallas_call(
        _fin_body,
        grid=(NBLK,),
        in_specs=[
            hspec, hspec, hspec, hspec,
            vspec, vspec, vspec, vspec,
            pl.BlockSpec((2, HH), lambda i: (0, 0)),
            pl.BlockSpec((2, HH, H), lambda i: (0, 0, 0)),
            pl.BlockSpec((1, H), lambda i: (0, 0)),
            pl.BlockSpec((H, O), lambda i: (0, 0)),
            pl.BlockSpec((1, O), lambda i: (0, 0)),
        ],
        out_specs=pl.BlockSpec((1, O), lambda i: (0, 0)),
        out_shape=jax.ShapeDtypeStruct((1, O), jnp.float32),
        scratch_shapes=[pltpu.VMEM((2, HH), jnp.float32)],
    )(a0, a1, g0, g1, dinv, dinvsq, cp0, cp1, b1r, W2r, b2, Wl, bl)


@jax.jit
def kernel(x, edge_index, W1, b1, W2, b2, W_lin, b_lin):
    ei = edge_index.astype(jnp.int32)
    src = ei[0]
    dst = ei[1]
    pad = EP - E
    src_p = jnp.concatenate([src, jnp.zeros((pad,), jnp.int32)])
    dst_p = jnp.concatenate([dst, jnp.full((pad,), SENT, jnp.int32)])
    dst2d_p = dst_p.reshape(EP // KB, KB)
    cnt0, cnt1 = _deg_kernel(dst2d_p)
    glo, ghi, dinv, dinvsq = _mm_call(
        x, W1, cnt0.reshape(NP, 1), cnt1.reshape(NP, 1),
    )
    dinvp = jnp.pad(dinv.reshape(N), (0, NP - N))
    cp0, cp1 = _cprime_kernel(src_p.reshape(EP // KB, KB), dst2d_p, dinvp)
    a_lo, a_hi = _edge_kernel(src_p.reshape(EP // KB, KB), dst2d_p, glo, ghi)
    return _fin_call(
        a_lo, a_hi, glo, ghi, dinv, dinvsq,
        cp0.reshape(NP, 1), cp1.reshape(NP, 1),
        b1.reshape(2, HH), W2.reshape(2, HH, H),
        b2.reshape(1, H), W_lin, b_lin.reshape(1, O),
    )
